# Initial kernel scaffold; baseline (speedup 1.0000x reference)
import jax
import jax.numpy as jnp
from jax.experimental import pallas as pl


def kernel(tensor, indexes):
    raise NotImplementedError("write your pallas kernel here")


# speedup vs baseline: 2.9775x; 2.9775x over previous
import functools

import jax
import jax.numpy as jnp
from jax import lax
from jax.experimental import pallas as pl
from jax.experimental.pallas import tpu as pltpu
from jax.experimental.pallas import tpu_sc as plsc

_INFO = plsc.get_sparse_core_info()
_NC, _NS = _INFO.num_cores, _INFO.num_subcores
_NW = _NC * _NS

_ROWS, _D = 100000, 128
_B0, _B1 = 4096, 50
_N = _B0 * _B1
_CHUNK = 128
_NCHUNKS = _N // _CHUNK
_CPW = _NCHUNKS // _NW


@functools.partial(
    pl.kernel,
    out_type=jax.ShapeDtypeStruct((_N, _D), jnp.float32),
    mesh=plsc.VectorSubcoreMesh(core_axis_name="c", subcore_axis_name="s"),
    scratch_types=[
        pltpu.VMEM((_CPW, _CHUNK), jnp.int32),
        pltpu.VMEM((_CHUNK, _D), jnp.float32),
        pltpu.SemaphoreType.DMA,
    ],
)
def _gather_sc(table_hbm, idx_hbm, out_hbm, idx_v, rows_v, gsem):
    wid = lax.axis_index("s") * _NC + lax.axis_index("c")
    base = wid * _CPW
    pltpu.sync_copy(idx_hbm.at[wid], idx_v)

    def body(j, carry):
        pltpu.async_copy(table_hbm.at[idx_v.at[j]], rows_v, gsem).wait()
        off = pl.multiple_of((base + j) * _CHUNK, _CHUNK)
        pltpu.sync_copy(rows_v, out_hbm.at[pl.ds(off, _CHUNK)])
        return carry

    lax.fori_loop(0, _CPW, body, 0)


def kernel(tensor, indexes):
    idx = indexes.astype(jnp.int32).reshape(_NW, _CPW, _CHUNK)
    out = _gather_sc(tensor, idx)
    return out.reshape(_B0, _B1, _D)


# speedup vs baseline: 3.3097x; 1.1116x over previous
import functools

import jax
import jax.numpy as jnp
from jax import lax
from jax.experimental import pallas as pl
from jax.experimental.pallas import tpu as pltpu
from jax.experimental.pallas import tpu_sc as plsc

_INFO = plsc.get_sparse_core_info()
_NC, _NS = _INFO.num_cores, _INFO.num_subcores
_NW = _NC * _NS

_ROWS, _D = 100000, 128
_B0, _B1 = 4096, 50
_N = _B0 * _B1
_CHUNK = 128
_NCHUNKS = _N // _CHUNK
_CPW = _NCHUNKS // _NW


_NBUF = 5
_NG = _CPW // _NBUF


@functools.partial(
    pl.kernel,
    out_type=jax.ShapeDtypeStruct((_N, _D), jnp.float32),
    mesh=plsc.VectorSubcoreMesh(core_axis_name="c", subcore_axis_name="s"),
    scratch_types=[
        pltpu.VMEM((_CPW, _CHUNK), jnp.int32),
        pltpu.VMEM((_NBUF, _CHUNK, _D), jnp.float32),
    ]
    + [pltpu.SemaphoreType.DMA] * (2 * _NBUF),
)
def _gather_sc(table_hbm, idx_hbm, out_hbm, idx_v, rows_v, *sems):
    gsems, wsems = sems[:_NBUF], sems[_NBUF:]
    wid = lax.axis_index("s") * _NC + lax.axis_index("c")
    base = wid * _CPW
    pltpu.sync_copy(idx_hbm.at[wid], idx_v)

    def out_slice(j):
        off = pl.multiple_of((base + j) * _CHUNK, _CHUNK)
        return out_hbm.at[pl.ds(off, _CHUNK)]

    for b in range(_NBUF):
        pltpu.async_copy(table_hbm.at[idx_v.at[b]], rows_v.at[b], gsems[b])

    def group(g, carry):
        for b in range(_NBUF):
            j = g * _NBUF + b
            pltpu.make_async_copy(
                table_hbm.at[idx_v.at[j]], rows_v.at[b], gsems[b]).wait()
            pltpu.async_copy(rows_v.at[b], out_slice(j), wsems[b])
        for b in range(_NBUF):
            jn = (g + 1) * _NBUF + b
            pltpu.make_async_copy(
                rows_v.at[b], out_slice(jn - _NBUF), wsems[b]).wait()
            pltpu.async_copy(table_hbm.at[idx_v.at[jn]], rows_v.at[b], gsems[b])
        return carry

    lax.fori_loop(0, _NG - 1, group, 0)

    last = (_NG - 1) * _NBUF
    for b in range(_NBUF):
        j = last + b
        pltpu.make_async_copy(
            table_hbm.at[idx_v.at[j]], rows_v.at[b], gsems[b]).wait()
        pltpu.async_copy(rows_v.at[b], out_slice(j), wsems[b])
    for b in range(_NBUF):
        pltpu.make_async_copy(
            rows_v.at[b], out_slice(last + b), wsems[b]).wait()


def kernel(tensor, indexes):
    idx = indexes.astype(jnp.int32).reshape(_NW, _CPW, _CHUNK)
    out = _gather_sc(tensor, idx)
    return out.reshape(_B0, _B1, _D)


# speedup vs baseline: 5.7977x; 1.7517x over previous
import functools

import jax
import jax.numpy as jnp
from jax import lax
from jax.experimental import pallas as pl
from jax.experimental.pallas import tpu as pltpu
from jax.experimental.pallas import tpu_sc as plsc

_INFO = plsc.get_sparse_core_info()
_NC, _NS = _INFO.num_cores, _INFO.num_subcores
_NW = _NC * _NS

_ROWS, _D = 100000, 128
_B0, _B1 = 4096, 50
_IPW = _B0 // _NW
_NI = 4
_NGRP = _IPW // _NI
_NBUF = 2
_NPAIR = _NGRP // _NBUF


@functools.partial(
    pl.kernel,
    out_type=jax.ShapeDtypeStruct((_B0, _B1, _D), jnp.float32),
    mesh=plsc.VectorSubcoreMesh(core_axis_name="c", subcore_axis_name="s"),
    scratch_types=[
        pltpu.VMEM((_IPW, _B1), jnp.int32),
        pltpu.VMEM((_NBUF, _NI, _B1, _D), jnp.float32),
    ]
    + [pltpu.SemaphoreType.DMA] * (2 * _NBUF),
)
def _gather_sc(table_hbm, idx_hbm, out_hbm, idx_v, rows_v, *sems):
    gsems, wsems = sems[:_NBUF], sems[_NBUF:]
    wid = lax.axis_index("s") * _NC + lax.axis_index("c")
    ibase = wid * _IPW
    pltpu.sync_copy(idx_hbm.at[pl.ds(pl.multiple_of(ibase, _IPW), _IPW)], idx_v)

    def gathers(g, b):
        return [
            pltpu.make_async_copy(
                table_hbm.at[idx_v.at[g * _NI + ii]], rows_v.at[b, ii], gsems[b])
            for ii in range(_NI)
        ]

    def write(g, b):
        i0 = pl.multiple_of(ibase + g * _NI, _NI)
        return pltpu.make_async_copy(
            rows_v.at[b], out_hbm.at[pl.ds(i0, _NI)], wsems[b])

    for b in range(_NBUF):
        for c in gathers(b, b):
            c.start()

    def pair(p, carry):
        g0 = p * _NBUF
        for b in range(_NBUF):
            for c in gathers(g0 + b, b):
                c.wait()
            write(g0 + b, b).start()
        for b in range(_NBUF):
            write(g0 + b, b).wait()
            for c in gathers(g0 + _NBUF + b, b):
                c.start()
        return carry

    lax.fori_loop(0, _NPAIR - 1, pair, 0)

    g0 = (_NPAIR - 1) * _NBUF
    for b in range(_NBUF):
        for c in gathers(g0 + b, b):
            c.wait()
        write(g0 + b, b).start()
    for b in range(_NBUF):
        write(g0 + b, b).wait()


def kernel(tensor, indexes):
    return _gather_sc(tensor, indexes.astype(jnp.int32))


# speedup vs baseline: 10.1275x; 1.7468x over previous
import functools

import jax
import jax.numpy as jnp
from jax import lax
from jax.experimental import pallas as pl
from jax.experimental.pallas import tpu as pltpu
from jax.experimental.pallas import tpu_sc as plsc

_INFO = plsc.get_sparse_core_info()
_NC, _NS = _INFO.num_cores, _INFO.num_subcores
_NW = _NC * _NS

_ROWS, _D = 100000, 128
_B0, _B1 = 4096, 50
_N = _B0 * _B1
_CHUNK = 128
_NCHUNKS = _N // _CHUNK
_CPW = _NCHUNKS // _NW
_NBUF = 5
_NG = _CPW // _NBUF


@functools.partial(
    pl.kernel,
    out_type=jax.ShapeDtypeStruct((_N, _D), jnp.float32),
    mesh=plsc.VectorSubcoreMesh(core_axis_name="c", subcore_axis_name="s"),
    scratch_types=[
        pltpu.VMEM((_CPW, _CHUNK), jnp.int32),
        pltpu.VMEM((_NBUF, _CHUNK, _D), jnp.float32),
    ]
    + [pltpu.SemaphoreType.DMA] * (2 * _NBUF),
)
def _gather_sc(table_hbm, idx_hbm, out_hbm, idx_v, rows_v, *sems):
    gsems, wsems = sems[:_NBUF], sems[_NBUF:]
    wid = lax.axis_index("s") * _NC + lax.axis_index("c")
    base = wid * _CPW
    pltpu.sync_copy(idx_hbm.at[wid], idx_v)

    def out_slice(j):
        off = pl.multiple_of((base + j) * _CHUNK, _CHUNK)
        return out_hbm.at[pl.ds(off, _CHUNK)]

    for b in range(_NBUF):
        pltpu.async_copy(table_hbm.at[idx_v.at[b]], rows_v.at[b], gsems[b])

    def group(g, carry):
        for b in range(_NBUF):
            j = g * _NBUF + b
            pltpu.make_async_copy(
                table_hbm.at[idx_v.at[j]], rows_v.at[b], gsems[b]).wait()
            pltpu.async_copy(rows_v.at[b], out_slice(j), wsems[b])
        for b in range(_NBUF):
            jn = (g + 1) * _NBUF + b
            pltpu.make_async_copy(
                rows_v.at[b], out_slice(jn - _NBUF), wsems[b]).wait()
            pltpu.async_copy(table_hbm.at[idx_v.at[jn]], rows_v.at[b], gsems[b])
        return carry

    lax.fori_loop(0, _NG - 1, group, 0)

    last = (_NG - 1) * _NBUF
    for b in range(_NBUF):
        j = last + b
        pltpu.make_async_copy(
            table_hbm.at[idx_v.at[j]], rows_v.at[b], gsems[b]).wait()
        pltpu.async_copy(rows_v.at[b], out_slice(j), wsems[b])
    for b in range(_NBUF):
        pltpu.make_async_copy(
            rows_v.at[b], out_slice(last + b), wsems[b]).wait()


def kernel(tensor, indexes):
    idx_t = indexes.astype(jnp.int32).T.reshape(_NW, _CPW, _CHUNK)
    out = _gather_sc(tensor, idx_t)
    return out.reshape(_B1, _B0, _D).transpose(1, 0, 2)
